# Initial kernel scaffold; baseline (speedup 1.0000x reference)
import jax
import jax.numpy as jnp
from jax.experimental import pallas as pl


def kernel(x, table):
    raise NotImplementedError("write your pallas kernel here")


# speedup vs baseline: 9.3133x; 9.3133x over previous
import functools

import jax
import jax.numpy as jnp
from jax import lax
from jax.experimental import pallas as pl
from jax.experimental.pallas import tpu as pltpu
from jax.experimental.pallas import tpu_sc as plsc

LANES = 16
ROW = 128
CHUNK = 1024
EXP = 3 * CHUNK


@functools.lru_cache(maxsize=None)
def _make_sc_gather(n: int, emb: int):
    info = plsc.get_sparse_core_info()
    nc, ns = info.num_cores, info.num_subcores
    nw = nc * ns
    per_w = n // nw
    n_chunks = per_w // CHUNK
    assert per_w * nw == n and n_chunks * CHUNK == per_w and emb == 3

    mesh = plsc.VectorSubcoreMesh(core_axis_name="c", subcore_axis_name="s")

    @functools.partial(
        pl.kernel,
        mesh=mesh,
        compiler_params=pltpu.CompilerParams(needs_layout_passes=False),
        out_type=jax.ShapeDtypeStruct((n * emb,), jnp.float32),
        scratch_types=[
            pltpu.VMEM((CHUNK,), jnp.int32),
            pltpu.VMEM((EXP,), jnp.int32),
            pltpu.VMEM((EXP,), jnp.float32),
            pltpu.SemaphoreType.DMA,
        ],
    )
    def k(idx_hbm, table_hbm, out_hbm, idx_v, gidx_v, rows_v, sem):
        wid = lax.axis_index("s") * nc + lax.axis_index("c")
        base = wid * per_w
        iota = lax.iota(jnp.int32, LANES)
        iota3 = iota * 3

        def body(c, carry):
            i0 = base + c * CHUNK
            pltpu.sync_copy(idx_hbm.at[pl.ds(i0, CHUNK)], idx_v)
            for m in range(CHUNK // LANES):
                a = idx_v[pl.ds(m * LANES, LANES)]
                b = a * 3
                s0 = iota3 + (m * 3 * LANES)
                plsc.store_scatter(gidx_v, [s0], b)
                plsc.store_scatter(gidx_v, [s0 + 1], b + 1)
                plsc.store_scatter(gidx_v, [s0 + 2], b + 2)
            copies = [
                pltpu.async_copy(
                    table_hbm.at[gidx_v.at[pl.ds(t * ROW, ROW)]],
                    rows_v.at[pl.ds(t * ROW, ROW)],
                    sem,
                )
                for t in range(EXP // ROW)
            ]
            for cp in copies:
                cp.wait()
            pltpu.sync_copy(rows_v, out_hbm.at[pl.ds(i0 * 3, EXP)])
            return carry

        lax.fori_loop(0, n_chunks, body, 0)

    return k


def kernel(x, table):
    b, s = x.shape
    vocab, emb = table.shape
    n = b * s
    out = _make_sc_gather(n, emb)(x.reshape(n), table.reshape(vocab * emb))
    return out.reshape(b, s, emb)
